# baseline (speedup 1.0000x reference)
import functools

import jax
import jax.numpy as jnp
from jax import lax
from jax.experimental import pallas as pl
from jax.experimental.pallas import tpu as pltpu

_BB = 4


def _body(months_ref, ch_ref, pos_ref, mtab_ref, tok_ref, out_ref):
    tok = tok_ref[...]
    bb, t = tok.shape[0], tok.shape[1]
    n = tok.shape[3] // 4
    m = months_ref[0]
    mo = jnp.zeros((bb, t, n), jnp.float32)
    for k in range(12):
        sel = (m == k).astype(jnp.float32)[..., None]
        mo = mo + sel * mtab_ref[k, :][None, None, :]
    ch = ch_ref[...]
    pos = pos_ref[...]
    out_ref[..., 0:n] = tok[..., 0:n] + ch[None, None, :, :]
    out_ref[..., n:2 * n] = tok[..., n:2 * n] + pos[None, :, None, :]
    out_ref[..., 2 * n:3 * n] = tok[..., 2 * n:3 * n] + mo[:, :, None, :]
    out_ref[..., 3 * n:] = tok[..., 3 * n:]


@jax.jit
def kernel(modality_tokens, timestamps, channel_embed, pos_embed, month_tab):
    b, t, bs, d = modality_tokens.shape
    months = timestamps[:, :, 1].astype(jnp.int32).reshape(b // _BB, _BB, t)
    grid = (b // _BB,)
    return pl.pallas_call(
        _body,
        grid=grid,
        in_specs=[
            pl.BlockSpec((1, _BB, t), lambda i: (i, 0, 0)),
            pl.BlockSpec((bs, d // 4), lambda i: (0, 0)),
            pl.BlockSpec((t, d // 4), lambda i: (0, 0)),
            pl.BlockSpec((12, d // 4), lambda i: (0, 0)),
            pl.BlockSpec((_BB, t, bs, d), lambda i: (i, 0, 0, 0)),
        ],
        out_specs=pl.BlockSpec((_BB, t, bs, d), lambda i: (i, 0, 0, 0)),
        out_shape=jax.ShapeDtypeStruct((b, t, bs, d), jnp.float32),
        compiler_params=pltpu.CompilerParams(
            dimension_semantics=("parallel",),
        ),
    )(months, channel_embed, pos_embed[:t], month_tab, modality_tokens)


# speedup vs baseline: 1.0929x; 1.0929x over previous
import jax
import jax.numpy as jnp
from jax.experimental import pallas as pl
from jax.experimental.pallas import tpu as pltpu

_CH = 4
_K = 4


def _body(months_ref, ch_ref, pos_ref, mtab_ref, tok_hbm, out_hbm,
          in_bufs, out_bufs, in_sems, out_sems):
    b, t, bs, d = tok_hbm.shape
    n = d // 4
    nchunk = b // _CH
    ch = ch_ref[...]
    pos = pos_ref[...]

    def in_copy(c):
        return pltpu.make_async_copy(
            tok_hbm.at[pl.ds(c * _CH, _CH)], in_bufs.at[c % _K],
            in_sems.at[c % _K])

    def out_copy(c):
        return pltpu.make_async_copy(
            out_bufs.at[c % _K], out_hbm.at[pl.ds(c * _CH, _CH)],
            out_sems.at[c % _K])

    for c in range(_K):
        in_copy(c).start()
    for c in range(nchunk):
        slot = c % _K
        in_copy(c).wait()
        if c >= _K:
            out_copy(c - _K).wait()
        tok = in_bufs[slot]
        m = months_ref[pl.ds(c * _CH, _CH), :]
        mo = jnp.zeros((_CH, t, n), jnp.float32)
        for k in range(12):
            sel = (m == k).astype(jnp.float32)[..., None]
            mo = mo + sel * mtab_ref[k, :][None, None, :]
        out_bufs[slot, ..., 0:n] = tok[..., 0:n] + ch[None, None, :, :]
        out_bufs[slot, ..., n:2 * n] = tok[..., n:2 * n] + pos[None, :, None, :]
        out_bufs[slot, ..., 2 * n:3 * n] = tok[..., 2 * n:3 * n] + mo[:, :, None, :]
        out_bufs[slot, ..., 3 * n:] = tok[..., 3 * n:]
        out_copy(c).start()
        if c + _K < nchunk:
            in_copy(c + _K).start()
    for c in range(nchunk - _K, nchunk):
        out_copy(c).wait()


@jax.jit
def kernel(modality_tokens, timestamps, channel_embed, pos_embed, month_tab):
    b, t, bs, d = modality_tokens.shape
    n = d // 4
    months = timestamps[:, :, 1].astype(jnp.int32)
    return pl.pallas_call(
        _body,
        in_specs=[
            pl.BlockSpec((b, t), lambda: (0, 0)),
            pl.BlockSpec((bs, n), lambda: (0, 0)),
            pl.BlockSpec((t, n), lambda: (0, 0)),
            pl.BlockSpec((12, n), lambda: (0, 0)),
            pl.BlockSpec(memory_space=pltpu.HBM),
        ],
        out_specs=pl.BlockSpec(memory_space=pltpu.HBM),
        out_shape=jax.ShapeDtypeStruct((b, t, bs, d), jnp.float32),
        scratch_shapes=[
            pltpu.VMEM((_K, _CH, t, bs, d), jnp.float32),
            pltpu.VMEM((_K, _CH, t, bs, d), jnp.float32),
            pltpu.SemaphoreType.DMA((_K,)),
            pltpu.SemaphoreType.DMA((_K,)),
        ],
        compiler_params=pltpu.CompilerParams(
            vmem_limit_bytes=100 * 1024 * 1024,
        ),
    )(months, channel_embed, pos_embed[:t], month_tab, modality_tokens)


# speedup vs baseline: 1.0947x; 1.0016x over previous
import jax
import jax.numpy as jnp
from jax.experimental import pallas as pl
from jax.experimental.pallas import tpu as pltpu

_CH = 4
_K = 4


def _body(months_ref, ch_ref, pos_ref, mtab_ref, tok_hbm, out_hbm, *scratch):
    in_bufs = scratch[0:_K]
    out_bufs = scratch[_K:2 * _K]
    in_sems = scratch[2 * _K:3 * _K]
    out_sems = scratch[3 * _K:4 * _K]
    b, t, bs, d = tok_hbm.shape
    n = d // 4
    nchunk = b // _CH
    ch = ch_ref[...]
    pos = pos_ref[...]

    def in_copy(c):
        return pltpu.make_async_copy(
            tok_hbm.at[pl.ds(c * _CH, _CH)], in_bufs[c % _K],
            in_sems[c % _K])

    def out_copy(c):
        return pltpu.make_async_copy(
            out_bufs[c % _K], out_hbm.at[pl.ds(c * _CH, _CH)],
            out_sems[c % _K])

    for c in range(_K):
        in_copy(c).start()
    for c in range(nchunk):
        slot = c % _K
        in_copy(c).wait()
        if c >= _K:
            out_copy(c - _K).wait()
        tok = in_bufs[slot][...]
        m = months_ref[pl.ds(c * _CH, _CH), :]
        mo = jnp.zeros((_CH, t, n), jnp.float32)
        for k in range(12):
            sel = (m == k).astype(jnp.float32)[..., None]
            mo = mo + sel * mtab_ref[k, :][None, None, :]
        out_bufs[slot][..., 0:n] = tok[..., 0:n] + ch[None, None, :, :]
        out_bufs[slot][..., n:2 * n] = tok[..., n:2 * n] + pos[None, :, None, :]
        out_bufs[slot][..., 2 * n:3 * n] = tok[..., 2 * n:3 * n] + mo[:, :, None, :]
        out_bufs[slot][..., 3 * n:] = tok[..., 3 * n:]
        out_copy(c).start()
        if c + _K < nchunk:
            in_copy(c + _K).start()
    for c in range(nchunk - _K, nchunk):
        out_copy(c).wait()


@jax.jit
def kernel(modality_tokens, timestamps, channel_embed, pos_embed, month_tab):
    b, t, bs, d = modality_tokens.shape
    n = d // 4
    months = timestamps[:, :, 1].astype(jnp.int32)
    scratch = (
        [pltpu.VMEM((_CH, t, bs, d), jnp.float32) for _ in range(2 * _K)]
        + [pltpu.SemaphoreType.DMA for _ in range(2 * _K)]
    )
    return pl.pallas_call(
        _body,
        in_specs=[
            pl.BlockSpec((b, t), lambda: (0, 0)),
            pl.BlockSpec((bs, n), lambda: (0, 0)),
            pl.BlockSpec((t, n), lambda: (0, 0)),
            pl.BlockSpec((12, n), lambda: (0, 0)),
            pl.BlockSpec(memory_space=pltpu.HBM),
        ],
        out_specs=pl.BlockSpec(memory_space=pltpu.HBM),
        out_shape=jax.ShapeDtypeStruct((b, t, bs, d), jnp.float32),
        scratch_shapes=scratch,
        compiler_params=pltpu.CompilerParams(
            vmem_limit_bytes=100 * 1024 * 1024,
        ),
    )(months, channel_embed, pos_embed[:t], month_tab, modality_tokens)
